# Initial kernel scaffold; baseline (speedup 1.0000x reference)
import jax
import jax.numpy as jnp
from jax.experimental import pallas as pl


def kernel(x_0, edge_index_0, batch_0, x_1, edge_index_1, batch_1, params):
    raise NotImplementedError("write your pallas kernel here")


# speedup vs baseline: 3.4975x; 3.4975x over previous
import functools

import jax
import jax.numpy as jnp
from jax import lax
from jax.experimental import pallas as pl
from jax.experimental.pallas import tpu as pltpu
from jax.experimental.pallas import tpu_sc as plsc

N = 50000
E = 800000
B = 512
NTILES = 16
NCORES = 2
R_FULL = 3128
R_LAST = N - 15 * R_FULL
ZCH = 184
EDGES_PER_TILE = E // NTILES
SW = 125
GROUP = 2
GROUP_EDGES = SW * GROUP
PHASE_GROUPS = 10
PHASE_EDGES = GROUP_EDGES * PHASE_GROUPS
NPHASES = EDGES_PER_TILE // PHASE_EDGES
PHASE_ROWS = PHASE_EDGES // SW
FC = 32


def _seg_kernel(x00, x01, x02, x10, x11, s0, d0, s1, d1,
                o00, o01, o02, o10, o11,
                accum, src_st, dst_st, rows0, rows1, gsem, ssem):
  cid = lax.axis_index("c")
  sid = lax.axis_index("s")
  tbase = sid * R_FULL

  def zero_accum():
    zv = jnp.zeros((16,), jnp.float32)
    def _zb(i, _):
      zbuf = rows0
      zbuf[i, pl.ds(0, 16)] = zv
      zbuf[i, pl.ds(16, 16)] = zv
      return 0
    lax.fori_loop(0, GROUP_EDGES, _zb, 0)

    nfull = jnp.where(sid == 15, 16, 17)
    def body(k, _):
      pltpu.sync_copy(rows0.at[pl.ds(0, ZCH)],
                      accum.at[pl.ds(tbase + k * ZCH, ZCH)])
      return 0
    lax.fori_loop(0, nfull, body, 0)

    @pl.when(sid == 15)
    def _():
      pltpu.sync_copy(rows0.at[pl.ds(0, 136)],
                      accum.at[pl.ds(tbase + 16 * ZCH, 136)])

  def edge_loop(x, src2d, dst2d):
    def phase(p, _):
      srow = sid * (NPHASES * PHASE_ROWS) + p * PHASE_ROWS
      pltpu.sync_copy(src2d.at[pl.ds(srow, PHASE_ROWS)], src_st)
      pltpu.sync_copy(dst2d.at[pl.ds(srow, PHASE_ROWS)], dst_st)

      def group(g, _):
        def run(buf):
          @pl.when(g >= 2)
          def _():
            pltpu.make_async_copy(x.at[pl.ds(0, GROUP_EDGES)], buf, ssem).wait()
          for j in range(GROUP):
            r = g * GROUP + j
            pltpu.async_copy(x.at[src_st.at[r]],
                             buf.at[pl.ds(j * SW, SW)], gsem)
          pltpu.make_async_copy(x.at[pl.ds(0, GROUP_EDGES)], buf, gsem).wait()
          for j in range(GROUP):
            r = g * GROUP + j
            pltpu.async_copy(buf.at[pl.ds(j * SW, SW)],
                             accum.at[dst_st.at[r]], ssem, add=True)

        @pl.when(g % 2 == 0)
        def _():
          run(rows0)

        @pl.when(g % 2 == 1)
        def _():
          run(rows1)
        return 0

      lax.fori_loop(0, PHASE_GROUPS, group, 0)
      pltpu.make_async_copy(x.at[pl.ds(0, GROUP_EDGES)], rows0, ssem).wait()
      pltpu.make_async_copy(x.at[pl.ds(0, GROUP_EDGES)], rows1, ssem).wait()
      return 0

    lax.fori_loop(0, NPHASES, phase, 0)

  def writeback(o):
    @pl.when(sid < 15)
    def _():
      pltpu.sync_copy(accum.at[pl.ds(tbase, R_FULL)],
                      o.at[pl.ds(tbase, R_FULL)])

    @pl.when(sid == 15)
    def _():
      pltpu.sync_copy(accum.at[pl.ds(tbase, R_LAST)],
                      o.at[pl.ds(tbase, R_LAST)])

  slots = [
      [(x00, s0, d0, o00), (x02, s0, d0, o02)],
      [(x01, s0, d0, o01), (x10, s1, d1, o10)],
      [(x11, s1, d1, o11), None],
  ]
  for slot in slots:
    for core in range(NCORES):
      if slot[core] is None:
        continue
      @pl.when(cid == core)
      def _(job=slot[core]):
        zero_accum()
    plsc.subcore_barrier()
    for core in range(NCORES):
      if slot[core] is None:
        continue
      @pl.when(cid == core)
      def _(job=slot[core]):
        edge_loop(job[0], job[1], job[2])
    plsc.subcore_barrier()
    for core in range(NCORES):
      if slot[core] is None:
        continue
      @pl.when(cid == core)
      def _(job=slot[core]):
        writeback(job[3])


_seg_sum = functools.partial(
    pl.kernel,
    out_type=[jax.ShapeDtypeStruct((N, FC), jnp.float32)] * 5,
    mesh=plsc.VectorSubcoreMesh(core_axis_name="c", subcore_axis_name="s",
                                num_cores=NCORES, num_subcores=NTILES),
    compiler_params=pltpu.CompilerParams(use_tc_tiling_on_sc=False),
    scratch_types=[
        pltpu.VMEM_SHARED((N, FC), jnp.float32),
        pltpu.VMEM((PHASE_ROWS, SW), jnp.int32),
        pltpu.VMEM((PHASE_ROWS, SW), jnp.int32),
        pltpu.VMEM((GROUP_EDGES, FC), jnp.float32),
        pltpu.VMEM((GROUP_EDGES, FC), jnp.float32),
        pltpu.SemaphoreType.DMA,
        pltpu.SemaphoreType.DMA,
    ],
)(_seg_kernel)


def _pad_cols(x, w):
  return jnp.pad(x, ((0, 0), (0, w - x.shape[1])))


def _seg_both(x0p, x1p, ei0, ei1):
  s0 = ei0[0].reshape(E // SW, SW)
  d0 = ei0[1].reshape(E // SW, SW)
  s1 = ei1[0].reshape(E // SW, SW)
  d1 = ei1[1].reshape(E // SW, SW)
  o = _seg_sum(x0p[:, 0:32], x0p[:, 32:64], x0p[:, 64:96],
               x1p[:, 0:32], x1p[:, 32:64], s0, d0, s1, d1)
  agg0 = jnp.concatenate(o[0:3], axis=1)
  agg1 = jnp.concatenate(o[3:5], axis=1)
  return agg0, agg1


def _pool_ref(x, batch, num_graphs):
  s = jax.ops.segment_sum(x, batch, num_segments=num_graphs)
  cnt = jax.ops.segment_sum(jnp.ones((x.shape[0],), x.dtype), batch,
                            num_segments=num_graphs)
  mean = s / jnp.maximum(cnt, 1.0)[:, None]
  mx = jax.ops.segment_max(x, batch, num_segments=num_graphs)
  return jnp.concatenate([mean, mx], axis=1)


def _pairnorm(x):
  xc = x - x.mean(axis=0, keepdims=True)
  return xc / jnp.sqrt(1e-5 + (xc ** 2).sum(axis=-1).mean())


def _bn(x, gamma, beta):
  m = x.mean(axis=0, keepdims=True)
  v = x.var(axis=0, keepdims=True)
  return (x - m) / jnp.sqrt(v + 1e-5) * gamma + beta


def kernel(x_0, edge_index_0, batch_0, x_1, edge_index_1, batch_1, params):
  p = params
  x0p = _pad_cols(x_0, 96)
  x1p = _pad_cols(x_1, 64)

  agg0, agg1 = _seg_both(x0p, x1p, edge_index_0, edge_index_1)

  h0 = jax.nn.relu((x_0 + agg0[:, :93]) @ p['conv1_W'] + p['conv1_b'])
  h1 = jax.nn.relu((x_1 + agg1[:, :43]) @ p['conv3_W'] + p['conv3_b'])

  agg0b, agg1b = _seg_both(_pad_cols(h0, 96), _pad_cols(h1, 64),
                           edge_index_0, edge_index_1)

  h0 = jax.nn.relu((h0 + agg0b[:, :93]) @ p['conv2_W'] + p['conv2_b'])
  h1 = jax.nn.relu((h1 + agg1b[:, :43]) @ p['conv4_W'] + p['conv4_b'])

  g0 = _pool_ref(h0, batch_0, B)
  g1 = _pool_ref(h1, batch_1, B)

  g0 = _pairnorm(g0)
  g0 = jax.nn.relu(g0 @ p['fcg0_W1'] + p['fcg0_b1'])
  g0 = _bn(g0 @ p['fcg0_W2'] + p['fcg0_b2'], p['fcg0_gamma'], p['fcg0_beta'])
  z0 = jax.nn.relu(g0 @ p['fcf0_W1'] + p['fcf0_b1'])
  z0 = jax.nn.sigmoid(z0 @ p['fcf0_W2'] + p['fcf0_b2'])

  g1 = _pairnorm(g1)
  g1 = jax.nn.relu(g1 @ p['fcg1_W1'] + p['fcg1_b1'])
  g1 = _bn(g1 @ p['fcg1_W2'] + p['fcg1_b2'], p['fcg1_gamma'], p['fcg1_beta'])
  z1 = jax.nn.relu(g1 @ p['fcf1_W1'] + p['fcf1_b1'])
  z1 = jax.nn.sigmoid(z1 @ p['fcf1_W2'] + p['fcf1_b2'])

  return (z0[:, 0], z0[:, 1], g0, g1, z1[:, 0], z1[:, 1])


# speedup vs baseline: 4.7207x; 1.3497x over previous
import functools

import jax
import jax.numpy as jnp
from jax import lax
from jax.experimental import pallas as pl
from jax.experimental.pallas import tpu as pltpu
from jax.experimental.pallas import tpu_sc as plsc

N = 50000
E = 800000
B = 512
NTILES = 16
NCORES = 2
R_FULL = 3128
R_LAST = N - 15 * R_FULL
ZCH = 184
EDGES_PER_TILE = E // NTILES
SW = 125
GROUP = 2
GROUP_EDGES = SW * GROUP
PHASE_GROUPS = 10
PHASE_EDGES = GROUP_EDGES * PHASE_GROUPS
NPHASES = EDGES_PER_TILE // PHASE_EDGES
PHASE_ROWS = PHASE_EDGES // SW
FC = 32


def _seg_kernel(x00, x01, x02, x10, x11, s0, d0, s1, d1,
                o00, o01, o02, o10, o11,
                accum, src_st, dst_st, rows0, rows1, gsem, ssem):
  cid = lax.axis_index("c")
  sid = lax.axis_index("s")
  tbase = sid * R_FULL

  def zero_accum():
    zv = jnp.zeros((16,), jnp.float32)
    def _zb(i, _):
      zbuf = rows0
      zbuf[i, pl.ds(0, 16)] = zv
      zbuf[i, pl.ds(16, 16)] = zv
      return 0
    lax.fori_loop(0, GROUP_EDGES, _zb, 0)

    nfull = jnp.where(sid == 15, 16, 17)
    def body(k, _):
      pltpu.sync_copy(rows0.at[pl.ds(0, ZCH)],
                      accum.at[pl.ds(tbase + k * ZCH, ZCH)])
      return 0
    lax.fori_loop(0, nfull, body, 0)

    @pl.when(sid == 15)
    def _():
      pltpu.sync_copy(rows0.at[pl.ds(0, 136)],
                      accum.at[pl.ds(tbase + 16 * ZCH, 136)])

  def edge_loop(x, src2d, dst2d):
    def phase(p, _):
      srow = sid * (NPHASES * PHASE_ROWS) + p * PHASE_ROWS
      pltpu.sync_copy(src2d.at[pl.ds(srow, PHASE_ROWS)], src_st)
      pltpu.sync_copy(dst2d.at[pl.ds(srow, PHASE_ROWS)], dst_st)

      def group(g, _):
        def run(buf):
          @pl.when(g >= 2)
          def _():
            pltpu.make_async_copy(x.at[pl.ds(0, GROUP_EDGES)], buf, ssem).wait()
          for j in range(GROUP):
            r = g * GROUP + j
            pltpu.async_copy(x.at[src_st.at[r]],
                             buf.at[pl.ds(j * SW, SW)], gsem)
          pltpu.make_async_copy(x.at[pl.ds(0, GROUP_EDGES)], buf, gsem).wait()
          for j in range(GROUP):
            r = g * GROUP + j
            pltpu.async_copy(buf.at[pl.ds(j * SW, SW)],
                             accum.at[dst_st.at[r]], ssem, add=True)

        @pl.when(g % 2 == 0)
        def _():
          run(rows0)

        @pl.when(g % 2 == 1)
        def _():
          run(rows1)
        return 0

      lax.fori_loop(0, PHASE_GROUPS, group, 0)
      pltpu.make_async_copy(x.at[pl.ds(0, GROUP_EDGES)], rows0, ssem).wait()
      pltpu.make_async_copy(x.at[pl.ds(0, GROUP_EDGES)], rows1, ssem).wait()
      return 0

    lax.fori_loop(0, NPHASES, phase, 0)

  def writeback(o):
    @pl.when(sid < 15)
    def _():
      pltpu.sync_copy(accum.at[pl.ds(tbase, R_FULL)],
                      o.at[pl.ds(tbase, R_FULL)])

    @pl.when(sid == 15)
    def _():
      pltpu.sync_copy(accum.at[pl.ds(tbase, R_LAST)],
                      o.at[pl.ds(tbase, R_LAST)])

  slots = [
      [(x00, s0, d0, o00), (x02, s0, d0, o02)],
      [(x01, s0, d0, o01), (x10, s1, d1, o10)],
      [(x11, s1, d1, o11), None],
  ]
  for slot in slots:
    for core in range(NCORES):
      if slot[core] is None:
        continue
      @pl.when(cid == core)
      def _(job=slot[core]):
        zero_accum()
    plsc.subcore_barrier()
    for core in range(NCORES):
      if slot[core] is None:
        continue
      @pl.when(cid == core)
      def _(job=slot[core]):
        edge_loop(job[0], job[1], job[2])
    plsc.subcore_barrier()
    for core in range(NCORES):
      if slot[core] is None:
        continue
      @pl.when(cid == core)
      def _(job=slot[core]):
        writeback(job[3])


_seg_sum = functools.partial(
    pl.kernel,
    out_type=[jax.ShapeDtypeStruct((N, FC), jnp.float32)] * 5,
    mesh=plsc.VectorSubcoreMesh(core_axis_name="c", subcore_axis_name="s",
                                num_cores=NCORES, num_subcores=NTILES),
    compiler_params=pltpu.CompilerParams(use_tc_tiling_on_sc=False),
    scratch_types=[
        pltpu.VMEM_SHARED((N, FC), jnp.float32),
        pltpu.VMEM((PHASE_ROWS, SW), jnp.int32),
        pltpu.VMEM((PHASE_ROWS, SW), jnp.int32),
        pltpu.VMEM((GROUP_EDGES, FC), jnp.float32),
        pltpu.VMEM((GROUP_EDGES, FC), jnp.float32),
        pltpu.SemaphoreType.DMA,
        pltpu.SemaphoreType.DMA,
    ],
)(_seg_kernel)


def _pad_cols(x, w):
  return jnp.pad(x, ((0, 0), (0, w - x.shape[1])))


_RB = 1000


def _gin_mm_body(n_chunks):
  def body(*refs):
    xs = refs[:n_chunks]
    aggs = refs[n_chunks:2 * n_chunks]
    w = refs[2 * n_chunks]
    b = refs[2 * n_chunks + 1]
    outs = refs[2 * n_chunks + 2:]
    x = jnp.concatenate([r[...] for r in xs], axis=1)
    a = jnp.concatenate([r[...] for r in aggs], axis=1)
    h = jax.nn.relu(
        jnp.dot(x + a, w[...], preferred_element_type=jnp.float32) + b[...])
    if len(outs) == 1:
      outs[0][...] = h
    else:
      for i, o in enumerate(outs):
        o[...] = h[:, i * FC:(i + 1) * FC]
  return body


def _gin_mm(x_chunks, agg_chunks, w, b, out_w):
  n = len(x_chunks)
  f = n * FC
  chunk_spec = pl.BlockSpec((_RB, FC), lambda i: (i, 0))
  if out_w is None:
    out_shape = [jax.ShapeDtypeStruct((N, FC), jnp.float32)] * n
    out_specs = [chunk_spec] * n
  else:
    out_shape = [jax.ShapeDtypeStruct((N, out_w), jnp.float32)]
    out_specs = [pl.BlockSpec((_RB, out_w), lambda i: (i, 0))]
  outs = pl.pallas_call(
      _gin_mm_body(n),
      grid=(N // _RB,),
      in_specs=[chunk_spec] * (2 * n)
      + [pl.BlockSpec(w.shape, lambda i: (0, 0)),
         pl.BlockSpec((1, w.shape[1]), lambda i: (0, 0))],
      out_specs=out_specs,
      out_shape=out_shape,
  )(*x_chunks, *agg_chunks, w, b.reshape(1, -1))
  return outs


def _mlp_body(mean_ref, mx_ref, w1a, w1b, b1, w2, b2, gamma, beta,
              wz1, bz1, wz2, bz2, g_out, z_out):
  mean = mean_ref[...]
  mx = mx_ref[...]
  cm = jnp.mean(mean, axis=0, keepdims=True)
  cx = jnp.mean(mx, axis=0, keepdims=True)
  xm = mean - cm
  xx = mx - cx
  ss = jnp.sum(xm * xm, axis=1) + jnp.sum(xx * xx, axis=1)
  s = 1.0 / jnp.sqrt(1e-5 + jnp.mean(ss))
  g = jnp.dot(xm * s, w1a[...], preferred_element_type=jnp.float32)
  g = g + jnp.dot(xx * s, w1b[...], preferred_element_type=jnp.float32)
  g = jax.nn.relu(g + b1[...])
  q = jnp.dot(g, w2[...], preferred_element_type=jnp.float32) + b2[...]
  m = jnp.mean(q, axis=0, keepdims=True)
  v = jnp.mean((q - m) * (q - m), axis=0, keepdims=True)
  gn = (q - m) / jnp.sqrt(v + 1e-5) * gamma[...] + beta[...]
  g_out[...] = gn
  z1 = jax.nn.relu(
      jnp.dot(gn, wz1[...], preferred_element_type=jnp.float32) + bz1[...])
  z_out[...] = jax.nn.sigmoid(
      jnp.dot(z1, wz2[...], preferred_element_type=jnp.float32) + bz2[...])


def _mlp(mean, mx, w1a, w1b, b1, w2, b2, gamma, beta, wz1, bz1, wz2, bz2):
  hp = mean.shape[1]
  full = lambda a: pl.BlockSpec(a.shape, lambda: (0,) * a.ndim)
  args = [mean, mx, w1a, w1b, b1.reshape(1, -1), w2, b2.reshape(1, -1),
          gamma.reshape(1, -1), beta.reshape(1, -1), wz1, bz1.reshape(1, -1),
          wz2, bz2.reshape(1, -1)]
  return pl.pallas_call(
      _mlp_body,
      in_specs=[full(a) for a in args],
      out_specs=[pl.BlockSpec((B, 512), lambda: (0, 0)),
                 pl.BlockSpec((B, 128), lambda: (0, 0))],
      out_shape=[jax.ShapeDtypeStruct((B, 512), jnp.float32),
                 jax.ShapeDtypeStruct((B, 128), jnp.float32)],
  )(*args)


def _seg_both(x0c, x1c, ei0, ei1):
  s0 = ei0[0].reshape(E // SW, SW)
  d0 = ei0[1].reshape(E // SW, SW)
  s1 = ei1[0].reshape(E // SW, SW)
  d1 = ei1[1].reshape(E // SW, SW)
  o = _seg_sum(x0c[0], x0c[1], x0c[2], x1c[0], x1c[1], s0, d0, s1, d1)
  return o[0:3], o[3:5]


def _pool_ref(x, batch, num_graphs):
  s = jax.ops.segment_sum(x, batch, num_segments=num_graphs)
  cnt = jax.ops.segment_sum(jnp.ones((x.shape[0],), x.dtype), batch,
                            num_segments=num_graphs)
  mean = s / jnp.maximum(cnt, 1.0)[:, None]
  mx = jax.ops.segment_max(x, batch, num_segments=num_graphs)
  return jnp.concatenate([mean, mx], axis=1)


def _pairnorm(x):
  xc = x - x.mean(axis=0, keepdims=True)
  return xc / jnp.sqrt(1e-5 + (xc ** 2).sum(axis=-1).mean())


def _bn(x, gamma, beta):
  m = x.mean(axis=0, keepdims=True)
  v = x.var(axis=0, keepdims=True)
  return (x - m) / jnp.sqrt(v + 1e-5) * gamma + beta


def _pad2(w, r, c):
  return jnp.pad(w, ((0, r - w.shape[0]), (0, c - w.shape[1])))


def kernel(x_0, edge_index_0, batch_0, x_1, edge_index_1, batch_1, params):
  p = params
  x0p = _pad_cols(x_0, 96)
  x1p = _pad_cols(x_1, 64)
  x0c = [x0p[:, i * FC:(i + 1) * FC] for i in range(3)]
  x1c = [x1p[:, i * FC:(i + 1) * FC] for i in range(2)]

  agg0, agg1 = _seg_both(x0c, x1c, edge_index_0, edge_index_1)

  w1p = _pad2(p['conv1_W'], 96, 96)
  h0c = _gin_mm(x0c, agg0, w1p, jnp.pad(p['conv1_b'], (0, 3)), None)
  w3p = _pad2(p['conv3_W'], 64, 64)
  h1c = _gin_mm(x1c, agg1, w3p, jnp.pad(p['conv3_b'], (0, 21)), None)

  agg0b, agg1b = _seg_both(h0c, h1c, edge_index_0, edge_index_1)

  w2p = _pad2(p['conv2_W'], 96, 1024)
  h2_0 = _gin_mm(h0c, agg0b, w2p, jnp.pad(p['conv2_b'], (0, 94)), 1024)[0]
  w4p = _pad2(p['conv4_W'], 64, 512)
  h2_1 = _gin_mm(h1c, agg1b, w4p, jnp.pad(p['conv4_b'], (0, 82)), 512)[0]

  g0cat = _pool_ref(h2_0[:, :930], batch_0, B)
  mean0 = _pad_cols(g0cat[:, :930], 1024)
  mx0 = _pad_cols(g0cat[:, 930:], 1024)
  g1cat = _pool_ref(h2_1[:, :430], batch_1, B)
  mean1 = _pad_cols(g1cat[:, :430], 512)
  mx1 = _pad_cols(g1cat[:, 430:], 512)

  g0, z0 = _mlp(mean0, mx0,
                _pad2(p['fcg0_W1'][:930], 1024, 1024),
                _pad2(p['fcg0_W1'][930:], 1024, 1024),
                p['fcg0_b1'], p['fcg0_W2'], p['fcg0_b2'],
                p['fcg0_gamma'], p['fcg0_beta'],
                p['fcf0_W1'], p['fcf0_b1'],
                _pad2(p['fcf0_W2'], 256, 128), jnp.pad(p['fcf0_b2'], (0, 126)))
  g1, z1 = _mlp(mean1, mx1,
                _pad2(p['fcg1_W1'][:430], 512, 1024),
                _pad2(p['fcg1_W1'][430:], 512, 1024),
                p['fcg1_b1'], p['fcg1_W2'], p['fcg1_b2'],
                p['fcg1_gamma'], p['fcg1_beta'],
                p['fcf1_W1'], p['fcf1_b1'],
                _pad2(p['fcf1_W2'], 256, 128), jnp.pad(p['fcf1_b2'], (0, 126)))

  return (z0[:, 0], z0[:, 1], g0, g1, z1[:, 0], z1[:, 1])


# speedup vs baseline: 5.5629x; 1.1784x over previous
import functools

import jax
import jax.numpy as jnp
from jax import lax
from jax.experimental import pallas as pl
from jax.experimental.pallas import tpu as pltpu
from jax.experimental.pallas import tpu_sc as plsc

N = 50000
E = 800000
B = 512
NTILES = 16
NCORES = 2
R_FULL = 3128
R_LAST = N - 15 * R_FULL
ZCH = 184
EDGES_PER_TILE = E // NTILES
SW = 125
GROUP = 2
GROUP_EDGES = SW * GROUP
PHASE_GROUPS = 10
PHASE_EDGES = GROUP_EDGES * PHASE_GROUPS
NPHASES = EDGES_PER_TILE // PHASE_EDGES
PHASE_ROWS = PHASE_EDGES // SW
FC = 32


def _seg_kernel(x00, x01, x02, x10, x11, s0, d0, s1, d1,
                o00, o01, o02, o10, o11,
                accum, src_st, dst_st, rows0, rows1, gsem, ssem):
  cid = lax.axis_index("c")
  sid = lax.axis_index("s")
  tbase = sid * R_FULL

  def zero_accum():
    zv = jnp.zeros((16,), jnp.float32)
    def _zb(i, _):
      zbuf = rows0
      zbuf[i, pl.ds(0, 16)] = zv
      zbuf[i, pl.ds(16, 16)] = zv
      return 0
    lax.fori_loop(0, GROUP_EDGES, _zb, 0)

    nfull = jnp.where(sid == 15, 16, 17)
    def body(k, _):
      pltpu.sync_copy(rows0.at[pl.ds(0, ZCH)],
                      accum.at[pl.ds(tbase + k * ZCH, ZCH)])
      return 0
    lax.fori_loop(0, nfull, body, 0)

    @pl.when(sid == 15)
    def _():
      pltpu.sync_copy(rows0.at[pl.ds(0, 136)],
                      accum.at[pl.ds(tbase + 16 * ZCH, 136)])

  def edge_loop(x, src2d, dst2d):
    def phase(p, _):
      srow = sid * (NPHASES * PHASE_ROWS) + p * PHASE_ROWS
      pltpu.sync_copy(src2d.at[pl.ds(srow, PHASE_ROWS)], src_st)
      pltpu.sync_copy(dst2d.at[pl.ds(srow, PHASE_ROWS)], dst_st)

      def group(g, _):
        def run(buf):
          @pl.when(g >= 2)
          def _():
            pltpu.make_async_copy(x.at[pl.ds(0, GROUP_EDGES)], buf, ssem).wait()
          for j in range(GROUP):
            r = g * GROUP + j
            pltpu.async_copy(x.at[src_st.at[r]],
                             buf.at[pl.ds(j * SW, SW)], gsem)
          pltpu.make_async_copy(x.at[pl.ds(0, GROUP_EDGES)], buf, gsem).wait()
          for j in range(GROUP):
            r = g * GROUP + j
            pltpu.async_copy(buf.at[pl.ds(j * SW, SW)],
                             accum.at[dst_st.at[r]], ssem, add=True)

        @pl.when(g % 2 == 0)
        def _():
          run(rows0)

        @pl.when(g % 2 == 1)
        def _():
          run(rows1)
        return 0

      lax.fori_loop(0, PHASE_GROUPS, group, 0)
      pltpu.make_async_copy(x.at[pl.ds(0, GROUP_EDGES)], rows0, ssem).wait()
      pltpu.make_async_copy(x.at[pl.ds(0, GROUP_EDGES)], rows1, ssem).wait()
      return 0

    lax.fori_loop(0, NPHASES, phase, 0)

  def writeback(o):
    @pl.when(sid < 15)
    def _():
      pltpu.sync_copy(accum.at[pl.ds(tbase, R_FULL)],
                      o.at[pl.ds(tbase, R_FULL)])

    @pl.when(sid == 15)
    def _():
      pltpu.sync_copy(accum.at[pl.ds(tbase, R_LAST)],
                      o.at[pl.ds(tbase, R_LAST)])

  slots = [
      [(x00, s0, d0, o00), (x02, s0, d0, o02)],
      [(x01, s0, d0, o01), (x10, s1, d1, o10)],
      [(x11, s1, d1, o11), None],
  ]
  for slot in slots:
    for core in range(NCORES):
      if slot[core] is None:
        continue
      @pl.when(cid == core)
      def _(job=slot[core]):
        zero_accum()
    plsc.subcore_barrier()
    for core in range(NCORES):
      if slot[core] is None:
        continue
      @pl.when(cid == core)
      def _(job=slot[core]):
        edge_loop(job[0], job[1], job[2])
    plsc.subcore_barrier()
    for core in range(NCORES):
      if slot[core] is None:
        continue
      @pl.when(cid == core)
      def _(job=slot[core]):
        writeback(job[3])


_seg_sum = functools.partial(
    pl.kernel,
    out_type=[jax.ShapeDtypeStruct((N, FC), jnp.float32)] * 5,
    mesh=plsc.VectorSubcoreMesh(core_axis_name="c", subcore_axis_name="s",
                                num_cores=NCORES, num_subcores=NTILES),
    compiler_params=pltpu.CompilerParams(use_tc_tiling_on_sc=False),
    scratch_types=[
        pltpu.VMEM_SHARED((N, FC), jnp.float32),
        pltpu.VMEM((PHASE_ROWS, SW), jnp.int32),
        pltpu.VMEM((PHASE_ROWS, SW), jnp.int32),
        pltpu.VMEM((GROUP_EDGES, FC), jnp.float32),
        pltpu.VMEM((GROUP_EDGES, FC), jnp.float32),
        pltpu.SemaphoreType.DMA,
        pltpu.SemaphoreType.DMA,
    ],
)(_seg_kernel)


def _pad_cols(x, w):
  return jnp.pad(x, ((0, 0), (0, w - x.shape[1])))


GPT = B // NTILES
CNT_FULL = 3136
CNT_LAST = N - 15 * CNT_FULL


def _pool_kernel(h0, h1, b0, b1, mean0, mx0, mean1, mx1,
                 cnts0, cnts1, batch_st, cnt_loc, cnt_all,
                 starts0, starts1, blkbuf, stg_mean, stg_max):
  cid = lax.axis_index("c")
  sid = lax.axis_index("s")
  g0 = sid * GPT
  ones = jnp.ones((16,), jnp.int32)
  zf = jnp.zeros((16,), jnp.float32)
  ninf = jnp.full((16,), -jnp.inf, jnp.float32)

  def local_counts(batch_hbm, cnts_sp):
    zi = jnp.zeros((16,), jnp.int32)
    def _z(i, _):
      cnt_loc[pl.ds(i * 16, 16)] = zi
      return 0
    lax.fori_loop(0, B // 16, _z, 0)

    @pl.when(sid < 15)
    def _():
      pltpu.sync_copy(batch_hbm.at[pl.ds(sid * CNT_FULL, CNT_FULL)],
                      batch_st.at[pl.ds(0, CNT_FULL)])

    @pl.when(sid == 15)
    def _():
      pltpu.sync_copy(batch_hbm.at[pl.ds(15 * CNT_FULL, CNT_LAST)],
                      batch_st.at[pl.ds(0, CNT_LAST)])
    ng = jnp.where(sid == 15, CNT_LAST // 16, CNT_FULL // 16)
    def _acc(i, _):
      ids = batch_st[pl.ds(i * 16, 16)]
      plsc.addupdate_scatter(cnt_loc, [ids], ones)
      return 0
    lax.fori_loop(0, ng, _acc, 0)
    pltpu.sync_copy(cnt_loc, cnts_sp.at[sid])

  def make_starts(cnts_sp, starts):
    pltpu.sync_copy(cnts_sp, cnt_all)
    def _pfx(i, carry):
      tot = jnp.zeros((16,), jnp.int32)
      def _row(r, t):
        return t + cnt_all[r, pl.ds(i * 16, 16)]
      tot = lax.fori_loop(0, NTILES, _row, tot)
      cs = plsc.cumsum(tot) + carry
      starts[pl.ds(i * 16, 16)] = cs - tot
      return cs[15]
    total = lax.fori_loop(0, B // 16, _pfx, jnp.zeros((), jnp.int32))
    starts[pl.ds(B, 16)] = jnp.full((16,), 1, jnp.int32) * total

  local_counts(b0, cnts0)
  local_counts(b1, cnts1)
  plsc.subcore_barrier()
  make_starts(cnts0, starts0)
  make_starts(cnts1, starts1)

  for h2, starts, mean_o, mx_o, width in (
      (h0, starts0, mean0, mx0, 1024),
      (h1, starts1, mean1, mx1, 512),
  ):
    ncc = width // 2 // 128

    def chunk_body(ci, _, h2=h2, starts=starts, mean_o=mean_o, mx_o=mx_o,
                   ncc=ncc, width=width):
      ccol = pl.multiple_of(cid * (width // 2) + ci * 128, 128)

      def graph_body(g, _):
        sv = starts[pl.ds(g, 16)]
        s = sv[0]
        e = sv[1]

        def blk_body(blk, carry):
          base = pl.multiple_of(jnp.minimum(blk * 128, N - 128), 8)
          pltpu.sync_copy(h2.at[pl.ds(base, 128), pl.ds(ccol, 128)], blkbuf)
          rlo = jnp.maximum(s, blk * 128)
          rhi = jnp.minimum(e, blk * 128 + 128)

          def row_body(r, c):
            loc = r - base
            out = []
            for k in range(8):
              v = blkbuf[loc, pl.ds(k * 16, 16)]
              out.append(c[k] + v)
              out.append(jnp.maximum(c[k + 8], v))
            return tuple(out[0::2]) + tuple(out[1::2])

          return lax.fori_loop(rlo, rhi, row_body, carry)

        init = (zf,) * 8 + (ninf,) * 8
        acc = lax.fori_loop(s // 128, (e + 127) // 128, blk_body, init)
        cntv = jnp.full((16,), 1.0, jnp.float32) * (e - s).astype(jnp.float32)
        rc = jnp.full((16,), 1.0, jnp.float32) / jnp.maximum(cntv, 1.0)
        for k in range(8):
          stg_mean[g - g0, pl.ds(k * 16, 16)] = acc[k] * rc
          stg_max[g - g0, pl.ds(k * 16, 16)] = acc[k + 8]
        return 0

      lax.fori_loop(g0, g0 + GPT, graph_body, 0)
      pltpu.sync_copy(stg_mean, mean_o.at[pl.ds(g0, GPT), pl.ds(ccol, 128)])
      pltpu.sync_copy(stg_max, mx_o.at[pl.ds(g0, GPT), pl.ds(ccol, 128)])
      return 0

    lax.fori_loop(0, ncc, chunk_body, 0)


_pool_sc = functools.partial(
    pl.kernel,
    out_type=[jax.ShapeDtypeStruct((B, 1024), jnp.float32),
              jax.ShapeDtypeStruct((B, 1024), jnp.float32),
              jax.ShapeDtypeStruct((B, 512), jnp.float32),
              jax.ShapeDtypeStruct((B, 512), jnp.float32)],
    mesh=plsc.VectorSubcoreMesh(core_axis_name="c", subcore_axis_name="s",
                                num_cores=NCORES, num_subcores=NTILES),
    compiler_params=pltpu.CompilerParams(needs_layout_passes=False),
    scratch_types=[
        pltpu.VMEM_SHARED((NTILES, B), jnp.int32),
        pltpu.VMEM_SHARED((NTILES, B), jnp.int32),
        pltpu.VMEM((CNT_FULL,), jnp.int32),
        pltpu.VMEM((B,), jnp.int32),
        pltpu.VMEM((NTILES, B), jnp.int32),
        pltpu.VMEM((B + 32,), jnp.int32),
        pltpu.VMEM((B + 32,), jnp.int32),
        pltpu.VMEM((128, 128), jnp.float32),
        pltpu.VMEM((GPT, 128), jnp.float32),
        pltpu.VMEM((GPT, 128), jnp.float32),
    ],
)(_pool_kernel)


_RB = 1000


def _gin_mm_body(n_chunks):
  def body(*refs):
    xs = refs[:n_chunks]
    aggs = refs[n_chunks:2 * n_chunks]
    w = refs[2 * n_chunks]
    b = refs[2 * n_chunks + 1]
    outs = refs[2 * n_chunks + 2:]
    x = jnp.concatenate([r[...] for r in xs], axis=1)
    a = jnp.concatenate([r[...] for r in aggs], axis=1)
    h = jax.nn.relu(
        jnp.dot(x + a, w[...], preferred_element_type=jnp.float32) + b[...])
    if len(outs) == 1:
      outs[0][...] = h
    else:
      for i, o in enumerate(outs):
        o[...] = h[:, i * FC:(i + 1) * FC]
  return body


def _gin_mm(x_chunks, agg_chunks, w, b, out_w):
  n = len(x_chunks)
  f = n * FC
  chunk_spec = pl.BlockSpec((_RB, FC), lambda i: (i, 0))
  if out_w is None:
    out_shape = [jax.ShapeDtypeStruct((N, FC), jnp.float32)] * n
    out_specs = [chunk_spec] * n
  else:
    out_shape = [jax.ShapeDtypeStruct((N, out_w), jnp.float32)]
    out_specs = [pl.BlockSpec((_RB, out_w), lambda i: (i, 0))]
  outs = pl.pallas_call(
      _gin_mm_body(n),
      grid=(N // _RB,),
      in_specs=[chunk_spec] * (2 * n)
      + [pl.BlockSpec(w.shape, lambda i: (0, 0)),
         pl.BlockSpec((1, w.shape[1]), lambda i: (0, 0))],
      out_specs=out_specs,
      out_shape=out_shape,
  )(*x_chunks, *agg_chunks, w, b.reshape(1, -1))
  return outs


def _mlp_body(mean_ref, mx_ref, w1a, w1b, b1, w2, b2, gamma, beta,
              wz1, bz1, wz2, bz2, g_out, z_out):
  mean = mean_ref[...]
  mx = mx_ref[...]
  cm = jnp.mean(mean, axis=0, keepdims=True)
  cx = jnp.mean(mx, axis=0, keepdims=True)
  xm = mean - cm
  xx = mx - cx
  ss = jnp.sum(xm * xm, axis=1) + jnp.sum(xx * xx, axis=1)
  s = 1.0 / jnp.sqrt(1e-5 + jnp.mean(ss))
  g = jnp.dot(xm * s, w1a[...], preferred_element_type=jnp.float32)
  g = g + jnp.dot(xx * s, w1b[...], preferred_element_type=jnp.float32)
  g = jax.nn.relu(g + b1[...])
  q = jnp.dot(g, w2[...], preferred_element_type=jnp.float32) + b2[...]
  m = jnp.mean(q, axis=0, keepdims=True)
  v = jnp.mean((q - m) * (q - m), axis=0, keepdims=True)
  gn = (q - m) / jnp.sqrt(v + 1e-5) * gamma[...] + beta[...]
  g_out[...] = gn
  z1 = jax.nn.relu(
      jnp.dot(gn, wz1[...], preferred_element_type=jnp.float32) + bz1[...])
  z_out[...] = jax.nn.sigmoid(
      jnp.dot(z1, wz2[...], preferred_element_type=jnp.float32) + bz2[...])


def _mlp(mean, mx, w1a, w1b, b1, w2, b2, gamma, beta, wz1, bz1, wz2, bz2):
  hp = mean.shape[1]
  full = lambda a: pl.BlockSpec(a.shape, lambda: (0,) * a.ndim)
  args = [mean, mx, w1a, w1b, b1.reshape(1, -1), w2, b2.reshape(1, -1),
          gamma.reshape(1, -1), beta.reshape(1, -1), wz1, bz1.reshape(1, -1),
          wz2, bz2.reshape(1, -1)]
  return pl.pallas_call(
      _mlp_body,
      in_specs=[full(a) for a in args],
      out_specs=[pl.BlockSpec((B, 512), lambda: (0, 0)),
                 pl.BlockSpec((B, 128), lambda: (0, 0))],
      out_shape=[jax.ShapeDtypeStruct((B, 512), jnp.float32),
                 jax.ShapeDtypeStruct((B, 128), jnp.float32)],
  )(*args)


def _seg_both(x0c, x1c, ei0, ei1):
  s0 = ei0[0].reshape(E // SW, SW)
  d0 = ei0[1].reshape(E // SW, SW)
  s1 = ei1[0].reshape(E // SW, SW)
  d1 = ei1[1].reshape(E // SW, SW)
  o = _seg_sum(x0c[0], x0c[1], x0c[2], x1c[0], x1c[1], s0, d0, s1, d1)
  return o[0:3], o[3:5]


def _pad2(w, r, c):
  return jnp.pad(w, ((0, r - w.shape[0]), (0, c - w.shape[1])))


def kernel(x_0, edge_index_0, batch_0, x_1, edge_index_1, batch_1, params):
  p = params
  x0p = _pad_cols(x_0, 96)
  x1p = _pad_cols(x_1, 64)
  x0c = [x0p[:, i * FC:(i + 1) * FC] for i in range(3)]
  x1c = [x1p[:, i * FC:(i + 1) * FC] for i in range(2)]

  agg0, agg1 = _seg_both(x0c, x1c, edge_index_0, edge_index_1)

  w1p = _pad2(p['conv1_W'], 96, 96)
  h0c = _gin_mm(x0c, agg0, w1p, jnp.pad(p['conv1_b'], (0, 3)), None)
  w3p = _pad2(p['conv3_W'], 64, 64)
  h1c = _gin_mm(x1c, agg1, w3p, jnp.pad(p['conv3_b'], (0, 21)), None)

  agg0b, agg1b = _seg_both(h0c, h1c, edge_index_0, edge_index_1)

  w2p = _pad2(p['conv2_W'], 96, 1024)
  h2_0 = _gin_mm(h0c, agg0b, w2p, jnp.pad(p['conv2_b'], (0, 94)), 1024)[0]
  w4p = _pad2(p['conv4_W'], 64, 512)
  h2_1 = _gin_mm(h1c, agg1b, w4p, jnp.pad(p['conv4_b'], (0, 82)), 512)[0]

  mean0, mx0, mean1, mx1 = _pool_sc(h2_0, h2_1, batch_0, batch_1)

  g0, z0 = _mlp(mean0, mx0,
                _pad2(p['fcg0_W1'][:930], 1024, 1024),
                _pad2(p['fcg0_W1'][930:], 1024, 1024),
                p['fcg0_b1'], p['fcg0_W2'], p['fcg0_b2'],
                p['fcg0_gamma'], p['fcg0_beta'],
                p['fcf0_W1'], p['fcf0_b1'],
                _pad2(p['fcf0_W2'], 256, 128), jnp.pad(p['fcf0_b2'], (0, 126)))
  g1, z1 = _mlp(mean1, mx1,
                _pad2(p['fcg1_W1'][:430], 512, 1024),
                _pad2(p['fcg1_W1'][430:], 512, 1024),
                p['fcg1_b1'], p['fcg1_W2'], p['fcg1_b2'],
                p['fcg1_gamma'], p['fcg1_beta'],
                p['fcf1_W1'], p['fcf1_b1'],
                _pad2(p['fcf1_W2'], 256, 128), jnp.pad(p['fcf1_b2'], (0, 126)))

  return (z0[:, 0], z0[:, 1], g0, g1, z1[:, 0], z1[:, 1])


# speedup vs baseline: 6.0558x; 1.0886x over previous
import functools

import jax
import jax.numpy as jnp
from jax import lax
from jax.experimental import pallas as pl
from jax.experimental.pallas import tpu as pltpu
from jax.experimental.pallas import tpu_sc as plsc

N = 50000
E = 800000
B = 512
NTILES = 16
NCORES = 2
R_FULL = 3128
R_LAST = N - 15 * R_FULL
ZCH = 184
EDGES_PER_TILE = E // NTILES
SW = 125
GROUP = 2
GROUP_EDGES = SW * GROUP
PHASE_GROUPS = 20
PHASE_EDGES = GROUP_EDGES * PHASE_GROUPS
NPHASES = EDGES_PER_TILE // PHASE_EDGES
PHASE_ROWS = PHASE_EDGES // SW
FC = 32


def _seg_kernel(x00, x01, x02, x10, x11, s0, d0, s1, d1,
                o00, o01, o02a, o02b, o10, o11,
                accum, src_st, dst_st, rows0, rows1, gsem, ssem):
  cid = lax.axis_index("c")
  sid = lax.axis_index("s")
  tbase = sid * R_FULL

  def zero_accum():
    zv = jnp.zeros((16,), jnp.float32)
    def _zb(i, _):
      zbuf = rows0
      zbuf[i, pl.ds(0, 16)] = zv
      zbuf[i, pl.ds(16, 16)] = zv
      return 0
    lax.fori_loop(0, GROUP_EDGES, _zb, 0)

    nfull = jnp.where(sid == 15, 16, 17)
    def body(k, _):
      pltpu.sync_copy(rows0.at[pl.ds(0, ZCH)],
                      accum.at[pl.ds(tbase + k * ZCH, ZCH)])
      return 0
    lax.fori_loop(0, nfull, body, 0)

    @pl.when(sid == 15)
    def _():
      pltpu.sync_copy(rows0.at[pl.ds(0, 136)],
                      accum.at[pl.ds(tbase + 16 * ZCH, 136)])

  def edge_loop(x, src2d, dst2d, base, nphases):
    rows_per_tile = nphases * PHASE_ROWS

    def fire_gathers(g, buf):
      for j in range(GROUP):
        pltpu.async_copy(x.at[src_st.at[g * GROUP + j]],
                         buf.at[pl.ds(j * SW, SW)], gsem)

    def fire_scatters(g, buf):
      for j in range(GROUP):
        pltpu.async_copy(buf.at[pl.ds(j * SW, SW)],
                         accum.at[dst_st.at[g * GROUP + j]], ssem, add=True)

    def drain(sem, buf):
      pltpu.make_async_copy(x.at[pl.ds(0, GROUP_EDGES)], buf, sem).wait()

    def phase(p, _):
      srow = base + sid * rows_per_tile + p * PHASE_ROWS
      pltpu.sync_copy(src2d.at[pl.ds(srow, PHASE_ROWS)], src_st)
      pltpu.sync_copy(dst2d.at[pl.ds(srow, PHASE_ROWS)], dst_st)

      def group(g, _):
        def run(buf, obuf):
          drain(gsem, buf)
          fire_scatters(g, buf)

          @pl.when(g >= 1)
          def _():
            drain(ssem, obuf)

          @pl.when(g + 1 < PHASE_GROUPS)
          def _():
            fire_gathers(g + 1, obuf)

        @pl.when(g % 2 == 0)
        def _():
          run(rows0, rows1)

        @pl.when(g % 2 == 1)
        def _():
          run(rows1, rows0)
        return 0

      fire_gathers(0, rows0)
      lax.fori_loop(0, PHASE_GROUPS, group, 0)
      drain(ssem, rows1)
      return 0

    lax.fori_loop(0, nphases, phase, 0)

  def writeback(o):
    @pl.when(sid < 15)
    def _():
      pltpu.sync_copy(accum.at[pl.ds(tbase, R_FULL)],
                      o.at[pl.ds(tbase, R_FULL)])

    @pl.when(sid == 15)
    def _():
      pltpu.sync_copy(accum.at[pl.ds(tbase, R_LAST)],
                      o.at[pl.ds(tbase, R_LAST)])

  slots = [
      [(x00, s0, d0, o00, 0, 10), (x10, s1, d1, o10, 0, 10)],
      [(x01, s0, d0, o01, 0, 10), (x11, s1, d1, o11, 0, 10)],
      [(x02, s0, d0, o02a, 0, 5), (x02, s0, d0, o02b, 3200, 5)],
  ]
  for slot in slots:
    for core in range(NCORES):
      @pl.when(cid == core)
      def _(job=slot[core]):
        zero_accum()
    plsc.subcore_barrier()
    for core in range(NCORES):
      @pl.when(cid == core)
      def _(job=slot[core]):
        edge_loop(job[0], job[1], job[2], job[4], job[5])
    plsc.subcore_barrier()
    for core in range(NCORES):
      @pl.when(cid == core)
      def _(job=slot[core]):
        writeback(job[3])


_seg_sum = functools.partial(
    pl.kernel,
    out_type=[jax.ShapeDtypeStruct((N, FC), jnp.float32)] * 6,
    mesh=plsc.VectorSubcoreMesh(core_axis_name="c", subcore_axis_name="s",
                                num_cores=NCORES, num_subcores=NTILES),
    compiler_params=pltpu.CompilerParams(use_tc_tiling_on_sc=False),
    scratch_types=[
        pltpu.VMEM_SHARED((N, FC), jnp.float32),
        pltpu.VMEM((PHASE_ROWS, SW), jnp.int32),
        pltpu.VMEM((PHASE_ROWS, SW), jnp.int32),
        pltpu.VMEM((GROUP_EDGES, FC), jnp.float32),
        pltpu.VMEM((GROUP_EDGES, FC), jnp.float32),
        pltpu.SemaphoreType.DMA,
        pltpu.SemaphoreType.DMA,
    ],
)(_seg_kernel)


def _pad_cols(x, w):
  return jnp.pad(x, ((0, 0), (0, w - x.shape[1])))


GPT = B // NTILES
CNT_FULL = 3136
CNT_LAST = N - 15 * CNT_FULL


def _pool_kernel(h0, h1, b0, b1, mean0, mx0, mean1, mx1,
                 cnts0, cnts1, batch_st, cnt_loc, cnt_all,
                 starts0, starts1, blkbuf, stg_mean, stg_max):
  cid = lax.axis_index("c")
  sid = lax.axis_index("s")
  g0 = sid * GPT
  ones = jnp.ones((16,), jnp.int32)
  zf = jnp.zeros((16,), jnp.float32)
  ninf = jnp.full((16,), -jnp.inf, jnp.float32)

  def local_counts(batch_hbm, cnts_sp):
    zi = jnp.zeros((16,), jnp.int32)
    def _z(i, _):
      cnt_loc[pl.ds(i * 16, 16)] = zi
      return 0
    lax.fori_loop(0, B // 16, _z, 0)

    @pl.when(sid < 15)
    def _():
      pltpu.sync_copy(batch_hbm.at[pl.ds(sid * CNT_FULL, CNT_FULL)],
                      batch_st.at[pl.ds(0, CNT_FULL)])

    @pl.when(sid == 15)
    def _():
      pltpu.sync_copy(batch_hbm.at[pl.ds(15 * CNT_FULL, CNT_LAST)],
                      batch_st.at[pl.ds(0, CNT_LAST)])
    ng = jnp.where(sid == 15, CNT_LAST // 16, CNT_FULL // 16)
    def _acc(i, _):
      ids = batch_st[pl.ds(i * 16, 16)]
      plsc.addupdate_scatter(cnt_loc, [ids], ones)
      return 0
    lax.fori_loop(0, ng, _acc, 0)
    pltpu.sync_copy(cnt_loc, cnts_sp.at[sid])

  def make_starts(cnts_sp, starts):
    pltpu.sync_copy(cnts_sp, cnt_all)
    def _pfx(i, carry):
      tot = jnp.zeros((16,), jnp.int32)
      def _row(r, t):
        return t + cnt_all[r, pl.ds(i * 16, 16)]
      tot = lax.fori_loop(0, NTILES, _row, tot)
      cs = plsc.cumsum(tot) + carry
      starts[pl.ds(i * 16, 16)] = cs - tot
      return cs[15]
    total = lax.fori_loop(0, B // 16, _pfx, jnp.zeros((), jnp.int32))
    starts[pl.ds(B, 16)] = jnp.full((16,), 1, jnp.int32) * total

  local_counts(b0, cnts0)
  local_counts(b1, cnts1)
  plsc.subcore_barrier()
  make_starts(cnts0, starts0)
  make_starts(cnts1, starts1)

  for h2, starts, mean_o, mx_o, width in (
      (h0, starts0, mean0, mx0, 1024),
      (h1, starts1, mean1, mx1, 512),
  ):
    ncc = width // 2 // 128

    def chunk_body(ci, _, h2=h2, starts=starts, mean_o=mean_o, mx_o=mx_o,
                   ncc=ncc, width=width):
      ccol = pl.multiple_of(cid * (width // 2) + ci * 128, 128)

      def graph_body(g, _):
        sv = starts[pl.ds(g, 16)]
        s = sv[0]
        e = sv[1]

        def blk_body(blk, carry):
          base = pl.multiple_of(jnp.minimum(blk * 128, N - 128), 8)
          pltpu.sync_copy(h2.at[pl.ds(base, 128), pl.ds(ccol, 128)], blkbuf)
          rlo = jnp.maximum(s, blk * 128)
          rhi = jnp.minimum(e, blk * 128 + 128)

          def row_body(r, c):
            loc = r - base
            out = []
            for k in range(8):
              v = blkbuf[loc, pl.ds(k * 16, 16)]
              out.append(c[k] + v)
              out.append(jnp.maximum(c[k + 8], v))
            return tuple(out[0::2]) + tuple(out[1::2])

          return lax.fori_loop(rlo, rhi, row_body, carry)

        init = (zf,) * 8 + (ninf,) * 8
        acc = lax.fori_loop(s // 128, (e + 127) // 128, blk_body, init)
        cntv = jnp.full((16,), 1.0, jnp.float32) * (e - s).astype(jnp.float32)
        rc = jnp.full((16,), 1.0, jnp.float32) / jnp.maximum(cntv, 1.0)
        for k in range(8):
          stg_mean[g - g0, pl.ds(k * 16, 16)] = acc[k] * rc
          stg_max[g - g0, pl.ds(k * 16, 16)] = acc[k + 8]
        return 0

      lax.fori_loop(g0, g0 + GPT, graph_body, 0)
      pltpu.sync_copy(stg_mean, mean_o.at[pl.ds(g0, GPT), pl.ds(ccol, 128)])
      pltpu.sync_copy(stg_max, mx_o.at[pl.ds(g0, GPT), pl.ds(ccol, 128)])
      return 0

    lax.fori_loop(0, ncc, chunk_body, 0)


_pool_sc = functools.partial(
    pl.kernel,
    out_type=[jax.ShapeDtypeStruct((B, 1024), jnp.float32),
              jax.ShapeDtypeStruct((B, 1024), jnp.float32),
              jax.ShapeDtypeStruct((B, 512), jnp.float32),
              jax.ShapeDtypeStruct((B, 512), jnp.float32)],
    mesh=plsc.VectorSubcoreMesh(core_axis_name="c", subcore_axis_name="s",
                                num_cores=NCORES, num_subcores=NTILES),
    compiler_params=pltpu.CompilerParams(needs_layout_passes=False),
    scratch_types=[
        pltpu.VMEM_SHARED((NTILES, B), jnp.int32),
        pltpu.VMEM_SHARED((NTILES, B), jnp.int32),
        pltpu.VMEM((CNT_FULL,), jnp.int32),
        pltpu.VMEM((B,), jnp.int32),
        pltpu.VMEM((NTILES, B), jnp.int32),
        pltpu.VMEM((B + 32,), jnp.int32),
        pltpu.VMEM((B + 32,), jnp.int32),
        pltpu.VMEM((128, 128), jnp.float32),
        pltpu.VMEM((GPT, 128), jnp.float32),
        pltpu.VMEM((GPT, 128), jnp.float32),
    ],
)(_pool_kernel)


_RB = 1000


def _gin_mm_body(n_chunks, n_aggs):
  def body(*refs):
    xs = refs[:n_chunks]
    aggs = refs[n_chunks:n_chunks + n_aggs]
    w = refs[n_chunks + n_aggs]
    b = refs[n_chunks + n_aggs + 1]
    outs = refs[n_chunks + n_aggs + 2:]
    x = jnp.concatenate([r[...] for r in xs], axis=1)
    av = [r[...] for r in aggs]
    if n_aggs > n_chunks:
      av = av[:n_chunks - 1] + [av[n_chunks - 1] + av[n_chunks]]
    a = jnp.concatenate(av, axis=1)
    h = jax.nn.relu(
        jnp.dot(x + a, w[...], preferred_element_type=jnp.float32) + b[...])
    if len(outs) == 1:
      outs[0][...] = h
    else:
      for i, o in enumerate(outs):
        o[...] = h[:, i * FC:(i + 1) * FC]
  return body


def _gin_mm(x_chunks, agg_chunks, w, b, out_w):
  n = len(x_chunks)
  na = len(agg_chunks)
  chunk_spec = pl.BlockSpec((_RB, FC), lambda i: (i, 0))
  if out_w is None:
    out_shape = [jax.ShapeDtypeStruct((N, FC), jnp.float32)] * n
    out_specs = [chunk_spec] * n
  else:
    out_shape = [jax.ShapeDtypeStruct((N, out_w), jnp.float32)]
    out_specs = [pl.BlockSpec((_RB, out_w), lambda i: (i, 0))]
  outs = pl.pallas_call(
      _gin_mm_body(n, na),
      grid=(N // _RB,),
      in_specs=[chunk_spec] * (n + na)
      + [pl.BlockSpec(w.shape, lambda i: (0, 0)),
         pl.BlockSpec((1, w.shape[1]), lambda i: (0, 0))],
      out_specs=out_specs,
      out_shape=out_shape,
  )(*x_chunks, *agg_chunks, w, b.reshape(1, -1))
  return outs


def _mlp_body(mean_ref, mx_ref, w1a, w1b, b1, w2, b2, gamma, beta,
              wz1, bz1, wz2, bz2, g_out, z_out):
  mean = mean_ref[...]
  mx = mx_ref[...]
  cm = jnp.mean(mean, axis=0, keepdims=True)
  cx = jnp.mean(mx, axis=0, keepdims=True)
  xm = mean - cm
  xx = mx - cx
  ss = jnp.sum(xm * xm, axis=1) + jnp.sum(xx * xx, axis=1)
  s = 1.0 / jnp.sqrt(1e-5 + jnp.mean(ss))
  g = jnp.dot(xm * s, w1a[...], preferred_element_type=jnp.float32)
  g = g + jnp.dot(xx * s, w1b[...], preferred_element_type=jnp.float32)
  g = jax.nn.relu(g + b1[...])
  q = jnp.dot(g, w2[...], preferred_element_type=jnp.float32) + b2[...]
  m = jnp.mean(q, axis=0, keepdims=True)
  v = jnp.mean((q - m) * (q - m), axis=0, keepdims=True)
  gn = (q - m) / jnp.sqrt(v + 1e-5) * gamma[...] + beta[...]
  g_out[...] = gn
  z1 = jax.nn.relu(
      jnp.dot(gn, wz1[...], preferred_element_type=jnp.float32) + bz1[...])
  z_out[...] = jax.nn.sigmoid(
      jnp.dot(z1, wz2[...], preferred_element_type=jnp.float32) + bz2[...])


def _mlp(mean, mx, w1a, w1b, b1, w2, b2, gamma, beta, wz1, bz1, wz2, bz2):
  hp = mean.shape[1]
  full = lambda a: pl.BlockSpec(a.shape, lambda: (0,) * a.ndim)
  args = [mean, mx, w1a, w1b, b1.reshape(1, -1), w2, b2.reshape(1, -1),
          gamma.reshape(1, -1), beta.reshape(1, -1), wz1, bz1.reshape(1, -1),
          wz2, bz2.reshape(1, -1)]
  return pl.pallas_call(
      _mlp_body,
      in_specs=[full(a) for a in args],
      out_specs=[pl.BlockSpec((B, 512), lambda: (0, 0)),
                 pl.BlockSpec((B, 128), lambda: (0, 0))],
      out_shape=[jax.ShapeDtypeStruct((B, 512), jnp.float32),
                 jax.ShapeDtypeStruct((B, 128), jnp.float32)],
  )(*args)


def _seg_both(x0c, x1c, ei0, ei1):
  s0 = ei0[0].reshape(E // SW, SW)
  d0 = ei0[1].reshape(E // SW, SW)
  s1 = ei1[0].reshape(E // SW, SW)
  d1 = ei1[1].reshape(E // SW, SW)
  o = _seg_sum(x0c[0], x0c[1], x0c[2], x1c[0], x1c[1], s0, d0, s1, d1)
  return o[0:4], o[4:6]


def _pad2(w, r, c):
  return jnp.pad(w, ((0, r - w.shape[0]), (0, c - w.shape[1])))


def kernel(x_0, edge_index_0, batch_0, x_1, edge_index_1, batch_1, params):
  p = params
  x0p = _pad_cols(x_0, 96)
  x1p = _pad_cols(x_1, 64)
  x0c = [x0p[:, i * FC:(i + 1) * FC] for i in range(3)]
  x1c = [x1p[:, i * FC:(i + 1) * FC] for i in range(2)]

  agg0, agg1 = _seg_both(x0c, x1c, edge_index_0, edge_index_1)

  w1p = _pad2(p['conv1_W'], 96, 96)
  h0c = _gin_mm(x0c, agg0, w1p, jnp.pad(p['conv1_b'], (0, 3)), None)
  w3p = _pad2(p['conv3_W'], 64, 64)
  h1c = _gin_mm(x1c, agg1, w3p, jnp.pad(p['conv3_b'], (0, 21)), None)

  agg0b, agg1b = _seg_both(h0c, h1c, edge_index_0, edge_index_1)

  w2p = _pad2(p['conv2_W'], 96, 1024)
  h2_0 = _gin_mm(h0c, agg0b, w2p, jnp.pad(p['conv2_b'], (0, 94)), 1024)[0]
  w4p = _pad2(p['conv4_W'], 64, 512)
  h2_1 = _gin_mm(h1c, agg1b, w4p, jnp.pad(p['conv4_b'], (0, 82)), 512)[0]

  mean0, mx0, mean1, mx1 = _pool_sc(h2_0, h2_1, batch_0, batch_1)

  g0, z0 = _mlp(mean0, mx0,
                _pad2(p['fcg0_W1'][:930], 1024, 1024),
                _pad2(p['fcg0_W1'][930:], 1024, 1024),
                p['fcg0_b1'], p['fcg0_W2'], p['fcg0_b2'],
                p['fcg0_gamma'], p['fcg0_beta'],
                p['fcf0_W1'], p['fcf0_b1'],
                _pad2(p['fcf0_W2'], 256, 128), jnp.pad(p['fcf0_b2'], (0, 126)))
  g1, z1 = _mlp(mean1, mx1,
                _pad2(p['fcg1_W1'][:430], 512, 1024),
                _pad2(p['fcg1_W1'][430:], 512, 1024),
                p['fcg1_b1'], p['fcg1_W2'], p['fcg1_b2'],
                p['fcg1_gamma'], p['fcg1_beta'],
                p['fcf1_W1'], p['fcf1_b1'],
                _pad2(p['fcf1_W2'], 256, 128), jnp.pad(p['fcf1_b2'], (0, 126)))

  return (z0[:, 0], z0[:, 1], g0, g1, z1[:, 0], z1[:, 1])


# speedup vs baseline: 6.2153x; 1.0263x over previous
import functools

import jax
import jax.numpy as jnp
from jax import lax
from jax.experimental import pallas as pl
from jax.experimental.pallas import tpu as pltpu
from jax.experimental.pallas import tpu_sc as plsc

N = 50000
E = 800000
B = 512
NTILES = 16
NCORES = 2
R_FULL = 3128
R_LAST = N - 15 * R_FULL
ZCH = 184
EDGES_PER_TILE = E // NTILES
SW = 125
GROUP = 2
GROUP_EDGES = SW * GROUP
PHASE_GROUPS = 20
PHASE_EDGES = GROUP_EDGES * PHASE_GROUPS
NPHASES = EDGES_PER_TILE // PHASE_EDGES
PHASE_ROWS = PHASE_EDGES // SW
FC = 32


def _seg_kernel(x00, x01, x02, x10, x11, s0, d0, s1, d1,
                o00, o01, o02a, o02b, o10, o11,
                accum, src_st, dst_st, rows0, rows1, gsem, ssem):
  cid = lax.axis_index("c")
  sid = lax.axis_index("s")
  tbase = sid * R_FULL

  def zero_accum():
    zv = jnp.zeros((16,), jnp.float32)
    def _zb(i, _):
      zbuf = rows0
      zbuf[i, pl.ds(0, 16)] = zv
      zbuf[i, pl.ds(16, 16)] = zv
      return 0
    lax.fori_loop(0, GROUP_EDGES, _zb, 0)

    nfull = jnp.where(sid == 15, 16, 17)
    def body(k, _):
      pltpu.sync_copy(rows0.at[pl.ds(0, ZCH)],
                      accum.at[pl.ds(tbase + k * ZCH, ZCH)])
      return 0
    lax.fori_loop(0, nfull, body, 0)

    @pl.when(sid == 15)
    def _():
      pltpu.sync_copy(rows0.at[pl.ds(0, 136)],
                      accum.at[pl.ds(tbase + 16 * ZCH, 136)])

  def edge_loop(x, src2d, dst2d, base, nphases):
    rows_per_tile = nphases * PHASE_ROWS

    def fire_gathers(g, buf):
      for j in range(GROUP):
        pltpu.async_copy(x.at[src_st.at[g * GROUP + j]],
                         buf.at[pl.ds(j * SW, SW)], gsem)

    def fire_scatters(g, buf):
      for j in range(GROUP):
        pltpu.async_copy(buf.at[pl.ds(j * SW, SW)],
                         accum.at[dst_st.at[g * GROUP + j]], ssem, add=True)

    def drain(sem, buf):
      pltpu.make_async_copy(x.at[pl.ds(0, GROUP_EDGES)], buf, sem).wait()

    def phase(p, _):
      srow = base + sid * rows_per_tile + p * PHASE_ROWS
      pltpu.sync_copy(src2d.at[pl.ds(srow, PHASE_ROWS)], src_st)
      pltpu.sync_copy(dst2d.at[pl.ds(srow, PHASE_ROWS)], dst_st)

      def group(g, _):
        def run(buf, obuf):
          drain(gsem, buf)
          fire_scatters(g, buf)

          @pl.when(g >= 1)
          def _():
            drain(ssem, obuf)

          @pl.when(g + 1 < PHASE_GROUPS)
          def _():
            fire_gathers(g + 1, obuf)

        @pl.when(g % 2 == 0)
        def _():
          run(rows0, rows1)

        @pl.when(g % 2 == 1)
        def _():
          run(rows1, rows0)
        return 0

      fire_gathers(0, rows0)
      lax.fori_loop(0, PHASE_GROUPS, group, 0)
      drain(ssem, rows1)
      return 0

    lax.fori_loop(0, nphases, phase, 0)

  def writeback(o):
    @pl.when(sid < 15)
    def _():
      pltpu.sync_copy(accum.at[pl.ds(tbase, R_FULL)],
                      o.at[pl.ds(tbase, R_FULL)])

    @pl.when(sid == 15)
    def _():
      pltpu.sync_copy(accum.at[pl.ds(tbase, R_LAST)],
                      o.at[pl.ds(tbase, R_LAST)])

  slots = [
      [(x00, s0, d0, o00, 0, 10), (x10, s1, d1, o10, 0, 10)],
      [(x01, s0, d0, o01, 0, 10), (x11, s1, d1, o11, 0, 10)],
      [(x02, s0, d0, o02a, 0, 5), (x02, s0, d0, o02b, 3200, 5)],
  ]
  for slot in slots:
    for core in range(NCORES):
      @pl.when(cid == core)
      def _(job=slot[core]):
        zero_accum()
    plsc.subcore_barrier()
    for core in range(NCORES):
      @pl.when(cid == core)
      def _(job=slot[core]):
        edge_loop(job[0], job[1], job[2], job[4], job[5])
    plsc.subcore_barrier()
    for core in range(NCORES):
      @pl.when(cid == core)
      def _(job=slot[core]):
        writeback(job[3])


_seg_sum = functools.partial(
    pl.kernel,
    out_type=[jax.ShapeDtypeStruct((N, FC), jnp.float32)] * 6,
    mesh=plsc.VectorSubcoreMesh(core_axis_name="c", subcore_axis_name="s",
                                num_cores=NCORES, num_subcores=NTILES),
    compiler_params=pltpu.CompilerParams(use_tc_tiling_on_sc=False),
    scratch_types=[
        pltpu.VMEM_SHARED((N, FC), jnp.float32),
        pltpu.VMEM((PHASE_ROWS, SW), jnp.int32),
        pltpu.VMEM((PHASE_ROWS, SW), jnp.int32),
        pltpu.VMEM((GROUP_EDGES, FC), jnp.float32),
        pltpu.VMEM((GROUP_EDGES, FC), jnp.float32),
        pltpu.SemaphoreType.DMA,
        pltpu.SemaphoreType.DMA,
    ],
)(_seg_kernel)


def _pad_cols(x, w):
  return jnp.pad(x, ((0, 0), (0, w - x.shape[1])))


GPT = B // NTILES
CNT_FULL = 3136
CNT_LAST = N - 15 * CNT_FULL


def _pool_kernel(h0, h1, b0, b1, mean0, mx0, mean1, mx1,
                 cnts0, cnts1, batch_st, cnt_loc, cnt_all,
                 starts0, starts1, batch_all, blkbuf0, blkbuf1,
                 stg_mean, stg_max, bsem):
  cid = lax.axis_index("c")
  sid = lax.axis_index("s")
  g0 = sid * GPT
  ones = jnp.ones((16,), jnp.int32)
  zf = jnp.zeros((16,), jnp.float32)
  ninf = jnp.full((16,), -jnp.inf, jnp.float32)

  def local_counts(batch_hbm, cnts_sp):
    zi = jnp.zeros((16,), jnp.int32)
    def _z(i, _):
      cnt_loc[pl.ds(i * 16, 16)] = zi
      return 0
    lax.fori_loop(0, B // 16, _z, 0)

    @pl.when(sid < 15)
    def _():
      pltpu.sync_copy(batch_hbm.at[pl.ds(sid * CNT_FULL, CNT_FULL)],
                      batch_st.at[pl.ds(0, CNT_FULL)])

    @pl.when(sid == 15)
    def _():
      pltpu.sync_copy(batch_hbm.at[pl.ds(15 * CNT_FULL, CNT_LAST)],
                      batch_st.at[pl.ds(0, CNT_LAST)])
    ng = jnp.where(sid == 15, CNT_LAST // 16, CNT_FULL // 16)
    def _acc(i, _):
      ids = batch_st[pl.ds(i * 16, 16)]
      plsc.addupdate_scatter(cnt_loc, [ids], ones)
      return 0
    lax.fori_loop(0, ng, _acc, 0)
    pltpu.sync_copy(cnt_loc, cnts_sp.at[sid])

  def make_starts(cnts_sp, starts):
    pltpu.sync_copy(cnts_sp, cnt_all)
    def _pfx(i, carry):
      tot = jnp.zeros((16,), jnp.int32)
      def _row(r, t):
        return t + cnt_all[r, pl.ds(i * 16, 16)]
      tot = lax.fori_loop(0, NTILES, _row, tot)
      cs = plsc.cumsum(tot) + carry
      starts[pl.ds(i * 16, 16)] = cs - tot
      return cs[15]
    total = lax.fori_loop(0, B // 16, _pfx, jnp.zeros((), jnp.int32))
    starts[pl.ds(B, 16)] = jnp.full((16,), 1, jnp.int32) * total

  local_counts(b0, cnts0)
  local_counts(b1, cnts1)
  plsc.subcore_barrier()
  make_starts(cnts0, starts0)
  make_starts(cnts1, starts1)

  for h2, starts, batch_hbm, mean_o, mx_o, width in (
      (h0, starts0, b0, mean0, mx0, 1024),
      (h1, starts1, b1, mean1, mx1, 512),
  ):
    ncc = width // 2 // 128
    pltpu.sync_copy(batch_hbm, batch_all.at[pl.ds(0, N)])
    sv = starts[pl.ds(g0, 16)]
    s_t = sv[0]
    ev = starts[pl.ds(g0 + GPT, 16)]
    e_t = ev[0]
    blk_lo = s_t // 128
    blk_hi = (e_t + 127) // 128

    def chunk_body(ci, _, h2=h2, starts=starts, mean_o=mean_o, mx_o=mx_o,
                   width=width, s_t=s_t, e_t=e_t, blk_lo=blk_lo,
                   blk_hi=blk_hi):
      ccol = pl.multiple_of(cid * (width // 2) + ci * 128, 128)

      def _init(i, _):
        for k in range(8):
          stg_mean[i, pl.ds(k * 16, 16)] = zf
          stg_max[i, pl.ds(k * 16, 16)] = ninf
        return 0
      lax.fori_loop(0, GPT, _init, 0)

      def stage(blk, buf):
        base = pl.multiple_of(jnp.minimum(blk * 128, N - 128), 8)
        pltpu.async_copy(h2.at[pl.ds(base, 128), pl.ds(ccol, 128)], buf, bsem)

      def rows(blk, carry, buf):
        base = pl.multiple_of(jnp.minimum(blk * 128, N - 128), 8)
        rlo = jnp.maximum(s_t, blk * 128)
        rhi = jnp.minimum(e_t, blk * 128 + 128)

        def row_body(r, c):
          gprev = c[0]
          loc = r - base
          g_r = batch_all[pl.ds(r, 16)][0]
          flush = g_r != gprev

          @pl.when(flush & (gprev >= 0))
          def _():
            row = gprev - g0
            for k in range(8):
              stg_mean[row, pl.ds(k * 16, 16)] = c[1 + k]
              stg_max[row, pl.ds(k * 16, 16)] = c[9 + k]
          out = [g_r]
          for k in range(8):
            v = buf[loc, pl.ds(k * 16, 16)]
            out.append(jnp.where(flush, v, c[1 + k] + v))
          for k in range(8):
            v = buf[loc, pl.ds(k * 16, 16)]
            out.append(jnp.where(flush, v, jnp.maximum(c[9 + k], v)))
          return tuple(out)

        return lax.fori_loop(rlo, rhi, row_body, carry)

      def blk_body(blk, carry):
        par = (blk - blk_lo) % 2

        @pl.when(blk + 1 < blk_hi)
        def _():
          @pl.when(par == 0)
          def _():
            stage(blk + 1, blkbuf1)

          @pl.when(par == 1)
          def _():
            stage(blk + 1, blkbuf0)

        return lax.cond(par == 0,
                        lambda c: rows(blk, c, blkbuf0),
                        lambda c: rows(blk, c, blkbuf1), carry)

      init = (jnp.full((), -1, jnp.int32),) + (zf,) * 8 + (ninf,) * 8

      @pl.when(blk_lo < blk_hi)
      def _():
        stage(blk_lo, blkbuf0)

      def blk_wrap(blk, carry):
        pltpu.make_async_copy(
            h2.at[pl.ds(0, 128), pl.ds(0, 128)], blkbuf0, bsem).wait()
        return blk_body(blk, carry)

      fin = lax.fori_loop(blk_lo, blk_hi, blk_wrap, init)

      @pl.when(fin[0] >= 0)
      def _():
        row = fin[0] - g0
        for k in range(8):
          stg_mean[row, pl.ds(k * 16, 16)] = fin[1 + k]
          stg_max[row, pl.ds(k * 16, 16)] = fin[9 + k]

      def _fins(i, _):
        svv = starts[pl.ds(g0 + i, 16)]
        cntv = jnp.full((16,), 1.0, jnp.float32) * (
            svv[1] - svv[0]).astype(jnp.float32)
        rc = jnp.full((16,), 1.0, jnp.float32) / jnp.maximum(cntv, 1.0)
        for k in range(8):
          stg_mean[i, pl.ds(k * 16, 16)] = stg_mean[i, pl.ds(k * 16, 16)] * rc
        return 0
      lax.fori_loop(0, GPT, _fins, 0)

      pltpu.sync_copy(stg_mean, mean_o.at[pl.ds(g0, GPT), pl.ds(ccol, 128)])
      pltpu.sync_copy(stg_max, mx_o.at[pl.ds(g0, GPT), pl.ds(ccol, 128)])
      return 0

    lax.fori_loop(0, ncc, chunk_body, 0)


_pool_sc = functools.partial(
    pl.kernel,
    out_type=[jax.ShapeDtypeStruct((B, 1024), jnp.float32),
              jax.ShapeDtypeStruct((B, 1024), jnp.float32),
              jax.ShapeDtypeStruct((B, 512), jnp.float32),
              jax.ShapeDtypeStruct((B, 512), jnp.float32)],
    mesh=plsc.VectorSubcoreMesh(core_axis_name="c", subcore_axis_name="s",
                                num_cores=NCORES, num_subcores=NTILES),
    compiler_params=pltpu.CompilerParams(needs_layout_passes=False),
    scratch_types=[
        pltpu.VMEM_SHARED((NTILES, B), jnp.int32),
        pltpu.VMEM_SHARED((NTILES, B), jnp.int32),
        pltpu.VMEM((CNT_FULL,), jnp.int32),
        pltpu.VMEM((B,), jnp.int32),
        pltpu.VMEM((NTILES, B), jnp.int32),
        pltpu.VMEM((B + 32,), jnp.int32),
        pltpu.VMEM((B + 32,), jnp.int32),
        pltpu.VMEM((N + 16,), jnp.int32),
        pltpu.VMEM((128, 128), jnp.float32),
        pltpu.VMEM((128, 128), jnp.float32),
        pltpu.VMEM((GPT, 128), jnp.float32),
        pltpu.VMEM((GPT, 128), jnp.float32),
        pltpu.SemaphoreType.DMA,
    ],
)(_pool_kernel)


_RB = 1000


def _gin_mm_body(n_chunks, n_aggs):
  def body(*refs):
    xs = refs[:n_chunks]
    aggs = refs[n_chunks:n_chunks + n_aggs]
    w = refs[n_chunks + n_aggs]
    b = refs[n_chunks + n_aggs + 1]
    outs = refs[n_chunks + n_aggs + 2:]
    x = jnp.concatenate([r[...] for r in xs], axis=1)
    av = [r[...] for r in aggs]
    if n_aggs > n_chunks:
      av = av[:n_chunks - 1] + [av[n_chunks - 1] + av[n_chunks]]
    a = jnp.concatenate(av, axis=1)
    h = jax.nn.relu(
        jnp.dot(x + a, w[...], preferred_element_type=jnp.float32) + b[...])
    if len(outs) == 1:
      outs[0][...] = h
    else:
      for i, o in enumerate(outs):
        o[...] = h[:, i * FC:(i + 1) * FC]
  return body


def _gin_mm(x_chunks, agg_chunks, w, b, out_w):
  n = len(x_chunks)
  na = len(agg_chunks)
  chunk_spec = pl.BlockSpec((_RB, FC), lambda i: (i, 0))
  if out_w is None:
    out_shape = [jax.ShapeDtypeStruct((N, FC), jnp.float32)] * n
    out_specs = [chunk_spec] * n
  else:
    out_shape = [jax.ShapeDtypeStruct((N, out_w), jnp.float32)]
    out_specs = [pl.BlockSpec((_RB, out_w), lambda i: (i, 0))]
  outs = pl.pallas_call(
      _gin_mm_body(n, na),
      grid=(N // _RB,),
      in_specs=[chunk_spec] * (n + na)
      + [pl.BlockSpec(w.shape, lambda i: (0, 0)),
         pl.BlockSpec((1, w.shape[1]), lambda i: (0, 0))],
      out_specs=out_specs,
      out_shape=out_shape,
  )(*x_chunks, *agg_chunks, w, b.reshape(1, -1))
  return outs


def _mlp_body(mean_ref, mx_ref, w1a, w1b, b1, w2, b2, gamma, beta,
              wz1, bz1, wz2, bz2, g_out, z_out):
  mean = mean_ref[...]
  mx = mx_ref[...]
  cm = jnp.mean(mean, axis=0, keepdims=True)
  cx = jnp.mean(mx, axis=0, keepdims=True)
  xm = mean - cm
  xx = mx - cx
  ss = jnp.sum(xm * xm, axis=1) + jnp.sum(xx * xx, axis=1)
  s = 1.0 / jnp.sqrt(1e-5 + jnp.mean(ss))
  g = jnp.dot(xm * s, w1a[...], preferred_element_type=jnp.float32)
  g = g + jnp.dot(xx * s, w1b[...], preferred_element_type=jnp.float32)
  g = jax.nn.relu(g + b1[...])
  q = jnp.dot(g, w2[...], preferred_element_type=jnp.float32) + b2[...]
  m = jnp.mean(q, axis=0, keepdims=True)
  v = jnp.mean((q - m) * (q - m), axis=0, keepdims=True)
  gn = (q - m) / jnp.sqrt(v + 1e-5) * gamma[...] + beta[...]
  g_out[...] = gn
  z1 = jax.nn.relu(
      jnp.dot(gn, wz1[...], preferred_element_type=jnp.float32) + bz1[...])
  z_out[...] = jax.nn.sigmoid(
      jnp.dot(z1, wz2[...], preferred_element_type=jnp.float32) + bz2[...])


def _mlp(mean, mx, w1a, w1b, b1, w2, b2, gamma, beta, wz1, bz1, wz2, bz2):
  hp = mean.shape[1]
  full = lambda a: pl.BlockSpec(a.shape, lambda: (0,) * a.ndim)
  args = [mean, mx, w1a, w1b, b1.reshape(1, -1), w2, b2.reshape(1, -1),
          gamma.reshape(1, -1), beta.reshape(1, -1), wz1, bz1.reshape(1, -1),
          wz2, bz2.reshape(1, -1)]
  return pl.pallas_call(
      _mlp_body,
      in_specs=[full(a) for a in args],
      out_specs=[pl.BlockSpec((B, 512), lambda: (0, 0)),
                 pl.BlockSpec((B, 128), lambda: (0, 0))],
      out_shape=[jax.ShapeDtypeStruct((B, 512), jnp.float32),
                 jax.ShapeDtypeStruct((B, 128), jnp.float32)],
  )(*args)


def _seg_both(x0c, x1c, ei0, ei1):
  s0 = ei0[0].reshape(E // SW, SW)
  d0 = ei0[1].reshape(E // SW, SW)
  s1 = ei1[0].reshape(E // SW, SW)
  d1 = ei1[1].reshape(E // SW, SW)
  o = _seg_sum(x0c[0], x0c[1], x0c[2], x1c[0], x1c[1], s0, d0, s1, d1)
  return o[0:4], o[4:6]


def _pad2(w, r, c):
  return jnp.pad(w, ((0, r - w.shape[0]), (0, c - w.shape[1])))


def kernel(x_0, edge_index_0, batch_0, x_1, edge_index_1, batch_1, params):
  p = params
  x0p = _pad_cols(x_0, 96)
  x1p = _pad_cols(x_1, 64)
  x0c = [x0p[:, i * FC:(i + 1) * FC] for i in range(3)]
  x1c = [x1p[:, i * FC:(i + 1) * FC] for i in range(2)]

  agg0, agg1 = _seg_both(x0c, x1c, edge_index_0, edge_index_1)

  w1p = _pad2(p['conv1_W'], 96, 96)
  h0c = _gin_mm(x0c, agg0, w1p, jnp.pad(p['conv1_b'], (0, 3)), None)
  w3p = _pad2(p['conv3_W'], 64, 64)
  h1c = _gin_mm(x1c, agg1, w3p, jnp.pad(p['conv3_b'], (0, 21)), None)

  agg0b, agg1b = _seg_both(h0c, h1c, edge_index_0, edge_index_1)

  w2p = _pad2(p['conv2_W'], 96, 1024)
  h2_0 = _gin_mm(h0c, agg0b, w2p, jnp.pad(p['conv2_b'], (0, 94)), 1024)[0]
  w4p = _pad2(p['conv4_W'], 64, 512)
  h2_1 = _gin_mm(h1c, agg1b, w4p, jnp.pad(p['conv4_b'], (0, 82)), 512)[0]

  mean0, mx0, mean1, mx1 = _pool_sc(h2_0, h2_1, batch_0, batch_1)

  g0, z0 = _mlp(mean0, mx0,
                _pad2(p['fcg0_W1'][:930], 1024, 1024),
                _pad2(p['fcg0_W1'][930:], 1024, 1024),
                p['fcg0_b1'], p['fcg0_W2'], p['fcg0_b2'],
                p['fcg0_gamma'], p['fcg0_beta'],
                p['fcf0_W1'], p['fcf0_b1'],
                _pad2(p['fcf0_W2'], 256, 128), jnp.pad(p['fcf0_b2'], (0, 126)))
  g1, z1 = _mlp(mean1, mx1,
                _pad2(p['fcg1_W1'][:430], 512, 1024),
                _pad2(p['fcg1_W1'][430:], 512, 1024),
                p['fcg1_b1'], p['fcg1_W2'], p['fcg1_b2'],
                p['fcg1_gamma'], p['fcg1_beta'],
                p['fcf1_W1'], p['fcf1_b1'],
                _pad2(p['fcf1_W2'], 256, 128), jnp.pad(p['fcf1_b2'], (0, 126)))

  return (z0[:, 0], z0[:, 1], g0, g1, z1[:, 0], z1[:, 1])
